# Initial kernel scaffold; baseline (speedup 1.0000x reference)
import jax
import jax.numpy as jnp
from jax.experimental import pallas as pl


def kernel(x, edge_index, W_l, b_l, W_r):
    raise NotImplementedError("write your pallas kernel here")


# speedup vs baseline: 11.0149x; 11.0149x over previous
import functools

import jax
import jax.numpy as jnp
from jax import lax
from jax.experimental import pallas as pl
from jax.experimental.pallas import tpu as pltpu
from jax.experimental.pallas import tpu_sc as plsc

N_NODES = 10000
N_EDGES = 320000
D_FEAT = 128
N_CLASSES = 16

NC = 2
NS = 16
NW = NC * NS
CHUNK = 128
K = 79
E_PAD = NW * K * CHUNK
W_AGG = 2 * N_CLASSES
N_SP = 10112
ROWS_PER_TILE = N_SP // NS


def _proj_body(x_ref, w2_ref, out_ref):
    y = jnp.dot(x_ref[...], w2_ref[...], preferred_element_type=jnp.float32)
    col = lax.broadcasted_iota(jnp.int32, y.shape, 1)
    out_ref[...] = y + jnp.where(col == N_CLASSES, 1.0, 0.0)


def _fin_body(p0_ref, p1_ref, x_ref, wr_ref, b_ref, out_ref):
    ssum = p0_ref[...] + p1_ref[...]
    agg = ssum[:, :N_CLASSES]
    cnt = ssum[:, N_CLASSES:N_CLASSES + 1]
    mean = agg / jnp.maximum(cnt, 1.0)
    z = mean + b_ref[...] + jnp.dot(x_ref[...], wr_ref[...],
                                    preferred_element_type=jnp.float32)
    z = jnp.maximum(z, 0.0)
    m = jnp.max(z, axis=1, keepdims=True)
    lse = m + jnp.log(jnp.sum(jnp.exp(z - m), axis=1, keepdims=True))
    out_ref[...] = z - lse


def _make_sc_kernel():
    mesh = plsc.VectorSubcoreMesh(core_axis_name="c", subcore_axis_name="s",
                                  num_cores=NC, num_subcores=NS)

    @functools.partial(
        pl.kernel,
        out_type=jax.ShapeDtypeStruct((NC, N_SP, W_AGG), jnp.float32),
        mesh=mesh,
        scratch_types=[
            pltpu.VMEM((K, CHUNK), jnp.int32),
            pltpu.VMEM((K, CHUNK), jnp.int32),
            pltpu.VMEM((CHUNK, W_AGG), jnp.float32),
            pltpu.VMEM((ROWS_PER_TILE, W_AGG), jnp.float32),
            pltpu.VMEM_SHARED((N_SP, W_AGG), jnp.float32),
            pltpu.SemaphoreType.DMA,
        ],
        compiler_params=pltpu.CompilerParams(use_tc_tiling_on_sc=False),
    )
    def sc_aggregate(src_hbm, dst_hbm, y2_hbm, zeros_hbm, out_hbm,
                     src_v, dst_v, rows_v, slab_v, agg_sh, sem):
        c = lax.axis_index("c")
        s = lax.axis_index("s")
        wid = s * NC + c
        row0 = s * ROWS_PER_TILE

        pltpu.sync_copy(zeros_hbm.at[pl.ds(row0, ROWS_PER_TILE)], slab_v)
        pltpu.sync_copy(slab_v, agg_sh.at[pl.ds(row0, ROWS_PER_TILE)])
        plsc.subcore_barrier()

        pltpu.sync_copy(src_hbm.at[wid], src_v)
        pltpu.sync_copy(dst_hbm.at[wid], dst_v)

        def chunk(j, carry):
            pltpu.async_copy(y2_hbm.at[src_v.at[j]], rows_v, sem).wait()
            pltpu.sync_copy(rows_v, agg_sh.at[dst_v.at[j]], add=True)
            return carry

        lax.fori_loop(0, K, chunk, 0)
        plsc.subcore_barrier()

        pltpu.sync_copy(agg_sh.at[pl.ds(row0, ROWS_PER_TILE)], slab_v)
        pltpu.sync_copy(slab_v, out_hbm.at[c, pl.ds(row0, ROWS_PER_TILE)])

    return sc_aggregate


_SC_AGGREGATE = _make_sc_kernel()


def kernel(x, edge_index, W_l, b_l, W_r):
    ei = edge_index.astype(jnp.int32)
    pad = E_PAD - N_EDGES
    src = jnp.concatenate([ei[0], jnp.zeros((pad,), jnp.int32)])
    dst = jnp.concatenate([ei[1], jnp.full((pad,), N_NODES, jnp.int32)])
    src3 = src.reshape(NW, K, CHUNK)
    dst3 = dst.reshape(NW, K, CHUNK)

    w2 = jnp.concatenate(
        [W_l.T, jnp.zeros((D_FEAT, W_AGG - N_CLASSES), jnp.float32)], axis=1)

    blk = 400
    y2 = pl.pallas_call(
        _proj_body,
        grid=(N_NODES // blk,),
        in_specs=[
            pl.BlockSpec((blk, D_FEAT), lambda i: (i, 0)),
            pl.BlockSpec((D_FEAT, W_AGG), lambda i: (0, 0)),
        ],
        out_specs=pl.BlockSpec((blk, W_AGG), lambda i: (i, 0)),
        out_shape=jax.ShapeDtypeStruct((N_NODES, W_AGG), jnp.float32),
    )(x, w2)

    zeros = jnp.zeros((N_SP, W_AGG), jnp.float32)
    parts = _SC_AGGREGATE(src3, dst3, y2, zeros)

    p0 = parts[0, :N_NODES, :]
    p1 = parts[1, :N_NODES, :]
    out = pl.pallas_call(
        _fin_body,
        grid=(N_NODES // blk,),
        in_specs=[
            pl.BlockSpec((blk, W_AGG), lambda i: (i, 0)),
            pl.BlockSpec((blk, W_AGG), lambda i: (i, 0)),
            pl.BlockSpec((blk, D_FEAT), lambda i: (i, 0)),
            pl.BlockSpec((D_FEAT, N_CLASSES), lambda i: (0, 0)),
            pl.BlockSpec((1, N_CLASSES), lambda i: (0, 0)),
        ],
        out_specs=pl.BlockSpec((blk, N_CLASSES), lambda i: (i, 0)),
        out_shape=jax.ShapeDtypeStruct((N_NODES, N_CLASSES), jnp.float32),
    )(p0, p1, x, W_r.T, b_l.reshape(1, N_CLASSES))
    return out
